# Initial kernel scaffold; baseline (speedup 1.0000x reference)
import jax
import jax.numpy as jnp
from jax.experimental import pallas as pl


def kernel(x, mixture_probs_pre_softmax, log_stds, mean_multiplier):
    raise NotImplementedError("write your pallas kernel here")


# speedup vs baseline: 10.2442x; 10.2442x over previous
import jax
import jax.numpy as jnp
from jax import lax
from jax.experimental import pallas as pl
from jax.experimental.pallas import tpu as pltpu
from jax.experimental.pallas import tpu_sc as plsc

_SM = 0.5
_K = 64
_N = 65536
_NC = 2
_NS = 16
_NW = _NC * _NS
_RPW = _N // _NW
_CHUNK = 256
_NCHUNK = _RPW // _CHUNK
_J = _K // 16


def _sc_body(x_hbm, pre_hbm, ls_hbm, mm_hbm, out_hbm, x_v, pre_v, ls_v, mm_v, out_v, sem):
    del sem
    pltpu.sync_copy(pre_hbm, pre_v)
    pltpu.sync_copy(ls_hbm, ls_v)
    pltpu.sync_copy(mm_hbm, mm_v)
    mm = mm_v[...]
    cs, qs, ms = [], [], []
    for j in range(_J):
        pre_j = pre_v[pl.ds(16 * j, 16)]
        ls_j = ls_v[pl.ds(16 * j, 16)]
        kf = lax.iota(jnp.int32, 16).astype(jnp.float32) + jnp.float32(16 * j)
        ms.append(mm * kf)
        cs.append(jnp.float32(_SM) * pre_j - ls_j)
        qs.append(jnp.float32(0.5) * jnp.exp(jnp.float32(-2.0) * ls_j))

    wid = lax.axis_index("s") * _NC + lax.axis_index("c")
    base = wid * _RPW

    def chunk_body(g, carry):
        rb = base + g * _CHUNK
        pltpu.sync_copy(x_hbm.at[pl.ds(rb, _CHUNK)], x_v)

        @plsc.parallel_loop(0, _CHUNK, 1, unroll=4)
        def _samp(i):
            idxv = jnp.full((16,), i, jnp.int32)
            xb = plsc.load_gather(x_v, [idxv])
            lps = []
            for j in range(_J):
                d = xb - ms[j]
                lps.append(cs[j] - d * d * qs[j])
            mx = jnp.max(jnp.maximum(jnp.maximum(lps[0], lps[1]),
                                     jnp.maximum(lps[2], lps[3])))
            es = [jnp.exp(lp - mx) for lp in lps]
            s = jnp.sum((es[0] + es[1]) + (es[2] + es[3]))
            r = jnp.ones((16,), jnp.float32) / lax.broadcast(s, (16,))
            off = i * _K
            for j in range(_J):
                out_v[pl.ds(off + 16 * j, 16)] = es[j] * r

        pltpu.sync_copy(out_v, out_hbm.at[pl.ds(rb * _K, _CHUNK * _K)])
        return carry

    lax.fori_loop(0, _NCHUNK, chunk_body, 0)


def kernel(x, mixture_probs_pre_softmax, log_stds, mean_multiplier):
    mm16 = jnp.broadcast_to(mean_multiplier.astype(jnp.float32), (16,))
    mesh = plsc.VectorSubcoreMesh(
        core_axis_name="c", subcore_axis_name="s",
        num_cores=_NC, num_subcores=_NS)
    out_flat = pl.kernel(
        _sc_body,
        out_type=jax.ShapeDtypeStruct((_N * _K,), jnp.float32),
        mesh=mesh,
        compiler_params=pltpu.CompilerParams(needs_layout_passes=False),
        scratch_types=[
            pltpu.VMEM((_CHUNK,), jnp.float32),
            pltpu.VMEM((_K,), jnp.float32),
            pltpu.VMEM((_K,), jnp.float32),
            pltpu.VMEM((16,), jnp.float32),
            pltpu.VMEM((_CHUNK * _K,), jnp.float32),
            pltpu.SemaphoreType.DMA,
        ],
    )(x, mixture_probs_pre_softmax, log_stds, mm16)
    return out_flat.reshape(_N, _K)


# speedup vs baseline: 10.2454x; 1.0001x over previous
import jax
import jax.numpy as jnp
from jax import lax
from jax.experimental import pallas as pl
from jax.experimental.pallas import tpu as pltpu
from jax.experimental.pallas import tpu_sc as plsc

_SM = 0.5
_K = 64
_N = 65536
_NC = 2
_NS = 16
_NW = _NC * _NS
_RPW = _N // _NW
_CHUNK = 256
_NCHUNK = _RPW // _CHUNK
_J = _K // 16


def _sc_body(x_hbm, pre_hbm, ls_hbm, mm_hbm, out_hbm, x_v, pre_v, ls_v, mm_v, out_v, sem):
    del sem
    pltpu.sync_copy(pre_hbm, pre_v)
    pltpu.sync_copy(ls_hbm, ls_v)
    pltpu.sync_copy(mm_hbm, mm_v)
    mm = mm_v[...]
    cs, qs, ms = [], [], []
    for j in range(_J):
        pre_j = pre_v[pl.ds(16 * j, 16)]
        ls_j = ls_v[pl.ds(16 * j, 16)]
        kf = lax.iota(jnp.int32, 16).astype(jnp.float32) + jnp.float32(16 * j)
        ms.append(mm * kf)
        cs.append(jnp.float32(_SM) * pre_j - ls_j)
        qs.append(jnp.float32(0.5) * jnp.exp(jnp.float32(-2.0) * ls_j))

    wid = lax.axis_index("s") * _NC + lax.axis_index("c")
    base = wid * _RPW

    def chunk_body(g, carry):
        rb = base + g * _CHUNK
        pltpu.sync_copy(x_hbm.at[pl.ds(rb, _CHUNK)], x_v)

        @plsc.parallel_loop(0, _CHUNK, 1, unroll=8)
        def _samp(i):
            idxv = jnp.full((16,), i, jnp.int32)
            xb = plsc.load_gather(x_v, [idxv])
            lps = []
            for j in range(_J):
                d = xb - ms[j]
                lps.append(cs[j] - d * d * qs[j])
            mx = jnp.max(jnp.maximum(jnp.maximum(lps[0], lps[1]),
                                     jnp.maximum(lps[2], lps[3])))
            es = [jnp.exp(lp - mx) for lp in lps]
            s = jnp.sum((es[0] + es[1]) + (es[2] + es[3]))
            r = jnp.ones((16,), jnp.float32) / lax.broadcast(s, (16,))
            off = i * _K
            for j in range(_J):
                out_v[pl.ds(off + 16 * j, 16)] = es[j] * r

        pltpu.sync_copy(out_v, out_hbm.at[pl.ds(rb * _K, _CHUNK * _K)])
        return carry

    lax.fori_loop(0, _NCHUNK, chunk_body, 0)


def kernel(x, mixture_probs_pre_softmax, log_stds, mean_multiplier):
    mm16 = jnp.broadcast_to(mean_multiplier.astype(jnp.float32), (16,))
    mesh = plsc.VectorSubcoreMesh(
        core_axis_name="c", subcore_axis_name="s",
        num_cores=_NC, num_subcores=_NS)
    out_flat = pl.kernel(
        _sc_body,
        out_type=jax.ShapeDtypeStruct((_N * _K,), jnp.float32),
        mesh=mesh,
        compiler_params=pltpu.CompilerParams(needs_layout_passes=False),
        scratch_types=[
            pltpu.VMEM((_CHUNK,), jnp.float32),
            pltpu.VMEM((_K,), jnp.float32),
            pltpu.VMEM((_K,), jnp.float32),
            pltpu.VMEM((16,), jnp.float32),
            pltpu.VMEM((_CHUNK * _K,), jnp.float32),
            pltpu.SemaphoreType.DMA,
        ],
    )(x, mixture_probs_pre_softmax, log_stds, mm16)
    return out_flat.reshape(_N, _K)


# speedup vs baseline: 12.1834x; 1.1892x over previous
import jax
import jax.numpy as jnp
from jax import lax
from jax.experimental import pallas as pl
from jax.experimental.pallas import tpu as pltpu
from jax.experimental.pallas import tpu_sc as plsc

_SM = 0.5
_K = 64
_N = 65536
_NC = 2
_NS = 16
_NW = _NC * _NS
_RPW = _N // _NW
_CHUNK = 256
_NCHUNK = _RPW // _CHUNK
_J = _K // 16


def _sc_body(x_hbm, pre_hbm, ls_hbm, mm_hbm, out_hbm, x_v, pre_v, ls_v, mm_v, out_v, sem):
    del sem
    pltpu.sync_copy(pre_hbm, pre_v)
    pltpu.sync_copy(ls_hbm, ls_v)
    pltpu.sync_copy(mm_hbm, mm_v)
    mm = mm_v[...]
    cs, qs, ms = [], [], []
    for j in range(_J):
        pre_j = pre_v[pl.ds(16 * j, 16)]
        ls_j = ls_v[pl.ds(16 * j, 16)]
        kf = lax.iota(jnp.int32, 16).astype(jnp.float32) + jnp.float32(16 * j)
        ms.append(mm * kf)
        cs.append(jnp.float32(_SM) * pre_j - ls_j)
        qs.append(jnp.float32(0.5) * jnp.exp(jnp.float32(-2.0) * ls_j))

    wid = lax.axis_index("s") * _NC + lax.axis_index("c")
    base = wid * _RPW

    def chunk_body(g, carry):
        rb = base + g * _CHUNK
        pltpu.sync_copy(x_hbm.at[pl.ds(rb, _CHUNK)], x_v)

        @plsc.parallel_loop(0, _CHUNK, 1, unroll=8)
        def _samp(i):
            idxv = jnp.full((16,), i, jnp.int32)
            xb = plsc.load_gather(x_v, [idxv])
            lps = []
            for j in range(_J):
                d = xb - ms[j]
                lps.append(cs[j] - d * d * qs[j])
            mx = jnp.max(jnp.maximum(jnp.maximum(lps[0], lps[1]),
                                     jnp.maximum(lps[2], lps[3])))
            es = [jnp.exp(lp - mx) for lp in lps]
            s = jnp.sum((es[0] + es[1]) + (es[2] + es[3]))
            r = jnp.ones((16,), jnp.float32) / lax.broadcast(s, (16,))
            for j in range(_J):
                out_v[i, pl.ds(16 * j, 16)] = es[j] * r

        pltpu.sync_copy(out_v, out_hbm.at[pl.ds(rb, _CHUNK)])
        return carry

    lax.fori_loop(0, _NCHUNK, chunk_body, 0)


def kernel(x, mixture_probs_pre_softmax, log_stds, mean_multiplier):
    mm16 = jnp.broadcast_to(mean_multiplier.astype(jnp.float32), (16,))
    mesh = plsc.VectorSubcoreMesh(
        core_axis_name="c", subcore_axis_name="s",
        num_cores=_NC, num_subcores=_NS)
    out_flat = pl.kernel(
        _sc_body,
        out_type=jax.ShapeDtypeStruct((_N, _K), jnp.float32),
        mesh=mesh,
        compiler_params=pltpu.CompilerParams(needs_layout_passes=False),
        scratch_types=[
            pltpu.VMEM((_CHUNK,), jnp.float32),
            pltpu.VMEM((_K,), jnp.float32),
            pltpu.VMEM((_K,), jnp.float32),
            pltpu.VMEM((16,), jnp.float32),
            pltpu.VMEM((_CHUNK, _K), jnp.float32),
            pltpu.SemaphoreType.DMA,
        ],
    )(x, mixture_probs_pre_softmax, log_stds, mm16)
    return out_flat
